# baseline (speedup 1.0000x reference)
import functools

import jax
import jax.numpy as jnp
from jax import lax
from jax.experimental import pallas as pl
from jax.experimental.pallas import tpu as pltpu
from jax.experimental.pallas import tpu_sc as plsc


_N = 8
_C = 192
_H = 64
_W = 64
_NWORKERS = 32
_PAIRS = _C // 2
_PAIRS_PER_WORKER = (_N * _PAIRS) // _NWORKERS
_NCHUNKS = _PAIRS_PER_WORKER
_NSLOTS = 6
_LAG = 3

_TC_CBLOCK = 32


def _make_sc_kernel(dtype):
    mesh = plsc.VectorSubcoreMesh(core_axis_name="c", subcore_axis_name="s")
    out_sds = jax.ShapeDtypeStruct((_N, _C, _H, _W), dtype)
    buf_t = pltpu.VMEM((_NSLOTS, 2, _H, _W), dtype)

    @functools.partial(
        pl.kernel,
        mesh=mesh,
        out_type=out_sds,
        scratch_types=[
            buf_t,
            pltpu.SemaphoreType.DMA((_NSLOTS,)),
            pltpu.SemaphoreType.DMA((_NSLOTS,)),
        ],
    )
    def sc_out1(x1_hbm, x2_hbm, o1_hbm, buf, sem_in, sem_out):
        wid = lax.axis_index("s") * 2 + lax.axis_index("c")
        workers_per_sample = _PAIRS // _PAIRS_PER_WORKER
        n = wid // workers_per_sample
        c0 = (wid % workers_per_sample) * 2 * _PAIRS_PER_WORKER

        def in_copies(k):
            slot = k % _NSLOTS
            c = c0 + 2 * k
            return (
                pltpu.make_async_copy(x2_hbm.at[n, c], buf.at[slot, 0], sem_in.at[slot]),
                pltpu.make_async_copy(x1_hbm.at[n, c + 1], buf.at[slot, 1], sem_in.at[slot]),
            )

        def out_copy(k):
            slot = k % _NSLOTS
            return pltpu.make_async_copy(
                buf.at[slot], o1_hbm.at[n, pl.ds(c0 + 2 * k, 2)], sem_out.at[slot])

        for k in range(min(_NSLOTS, _NCHUNKS)):
            for cp in in_copies(k):
                cp.start()

        for k in range(_NCHUNKS):
            for cp in in_copies(k):
                cp.wait()
            out_copy(k).start()
            j = k - _LAG
            if j >= 0 and j + _NSLOTS < _NCHUNKS:
                out_copy(j).wait()
                for cp in in_copies(j + _NSLOTS):
                    cp.start()

        lo = max(0, min(_NCHUNKS - _LAG, _NCHUNKS - _NSLOTS))
        for k in range(lo, _NCHUNKS):
            out_copy(k).wait()

    return sc_out1


def _tc_out2_body(x1_ref, x2_ref, o2_ref):
    ch = lax.broadcasted_iota(jnp.int32, (1, _TC_CBLOCK, 1, 1), 1)
    o2_ref[...] = jnp.where(ch % 2 == 0, x1_ref[...], x2_ref[...])


def _make_tc_kernel(dtype):
    spec = pl.BlockSpec((1, _TC_CBLOCK, _H, _W), lambda n, b: (n, b, 0, 0))
    return pl.pallas_call(
        _tc_out2_body,
        grid=(_N, _C // _TC_CBLOCK),
        in_specs=[spec, spec],
        out_specs=spec,
        out_shape=jax.ShapeDtypeStruct((_N, _C, _H, _W), dtype),
    )


def kernel(x1, x2):
    o1 = _make_sc_kernel(x1.dtype)(x1, x2)
    o2 = _make_tc_kernel(x1.dtype)(x1, x2)
    return (o1, o2)


# speedup vs baseline: 1.0435x; 1.0435x over previous
import functools

import jax
import jax.numpy as jnp
from jax import lax
from jax.experimental import pallas as pl
from jax.experimental.pallas import tpu as pltpu
from jax.experimental.pallas import tpu_sc as plsc


_N = 8
_C = 192
_H = 64
_W = 64
_NWORKERS = 32
_PAIRS = _C // 2
_PAIRS_PER_WORKER = (_N * _PAIRS) // _NWORKERS
_NCHUNKS = _PAIRS_PER_WORKER
_NSLOTS = 3
_LAG = 1


def _make_sc_kernel(dtype):
    mesh = plsc.VectorSubcoreMesh(core_axis_name="c", subcore_axis_name="s")
    out_sds = jax.ShapeDtypeStruct((_N, _C, _H, _W), dtype)
    buf_t = pltpu.VMEM((_NSLOTS, 2, _H, _W), dtype)

    @functools.partial(
        pl.kernel,
        mesh=mesh,
        out_type=[out_sds, out_sds],
        scratch_types=[
            buf_t,
            buf_t,
            pltpu.SemaphoreType.DMA((_NSLOTS,)),
            pltpu.SemaphoreType.DMA((_NSLOTS,)),
        ],
    )
    def sc_exchange(x1_hbm, x2_hbm, o1_hbm, o2_hbm, buf_a, buf_b, sem_in, sem_out):
        wid = lax.axis_index("s") * 2 + lax.axis_index("c")
        workers_per_sample = _PAIRS // _PAIRS_PER_WORKER
        n = wid // workers_per_sample
        c0 = (wid % workers_per_sample) * 2 * _PAIRS_PER_WORKER

        def in_copies(k):
            slot = k % _NSLOTS
            c = c0 + 2 * k
            return (
                pltpu.make_async_copy(x1_hbm.at[n, pl.ds(c, 2)], buf_a.at[slot], sem_in.at[slot]),
                pltpu.make_async_copy(x2_hbm.at[n, pl.ds(c, 2)], buf_b.at[slot], sem_in.at[slot]),
            )

        def out_copies(k):
            slot = k % _NSLOTS
            c = c0 + 2 * k
            return (
                pltpu.make_async_copy(buf_b.at[slot, 0], o1_hbm.at[n, c], sem_out.at[slot]),
                pltpu.make_async_copy(buf_a.at[slot, 1], o1_hbm.at[n, c + 1], sem_out.at[slot]),
                pltpu.make_async_copy(buf_a.at[slot, 0], o2_hbm.at[n, c], sem_out.at[slot]),
                pltpu.make_async_copy(buf_b.at[slot, 1], o2_hbm.at[n, c + 1], sem_out.at[slot]),
            )

        for k in range(min(_NSLOTS, _NCHUNKS)):
            for cp in in_copies(k):
                cp.start()

        for k in range(_NCHUNKS):
            for cp in in_copies(k):
                cp.wait()
            for cp in out_copies(k):
                cp.start()
            j = k - _LAG
            if j >= 0 and j + _NSLOTS < _NCHUNKS:
                for cp in out_copies(j):
                    cp.wait()
                for cp in in_copies(j + _NSLOTS):
                    cp.start()

        lo = max(0, min(_NCHUNKS - _LAG, _NCHUNKS - _NSLOTS))
        for k in range(lo, _NCHUNKS):
            for cp in out_copies(k):
                cp.wait()

    return sc_exchange


def kernel(x1, x2):
    o1, o2 = _make_sc_kernel(x1.dtype)(x1, x2)
    return (o1, o2)


# speedup vs baseline: 1.3944x; 1.3363x over previous
import functools

import jax
import jax.numpy as jnp
from jax import lax
from jax.experimental import pallas as pl
from jax.experimental.pallas import tpu as pltpu
from jax.experimental.pallas import tpu_sc as plsc


_N = 8
_CPAIRS = 96
_H = 64
_W = 64
_NWORKERS = 32
_PAIRS_PER_WORKER = (_N * _CPAIRS) // _NWORKERS
_NCHUNKS = _PAIRS_PER_WORKER
_NSLOTS = 6
_LAG = 3

_TC_BLOCK = 16


def _make_sc_kernel(dtype):
    mesh = plsc.VectorSubcoreMesh(core_axis_name="c", subcore_axis_name="s")
    out_sds = jax.ShapeDtypeStruct((_N, _CPAIRS, 2, _H, _W), dtype)
    buf_t = pltpu.VMEM((_NSLOTS, 2, _H, _W), dtype)

    @functools.partial(
        pl.kernel,
        mesh=mesh,
        out_type=out_sds,
        scratch_types=[
            buf_t,
            pltpu.SemaphoreType.DMA((_NSLOTS,)),
            pltpu.SemaphoreType.DMA((_NSLOTS,)),
        ],
    )
    def sc_out1(x1_hbm, x2_hbm, o1_hbm, buf, sem_in, sem_out):
        wid = lax.axis_index("s") * 2 + lax.axis_index("c")
        workers_per_sample = _CPAIRS // _PAIRS_PER_WORKER
        n = wid // workers_per_sample
        p0 = (wid % workers_per_sample) * _PAIRS_PER_WORKER

        def in_copies(k):
            slot = k % _NSLOTS
            p = p0 + k
            return (
                pltpu.make_async_copy(x2_hbm.at[n, p, 0], buf.at[slot, 0], sem_in.at[slot]),
                pltpu.make_async_copy(x1_hbm.at[n, p, 1], buf.at[slot, 1], sem_in.at[slot]),
            )

        def out_copy(k):
            slot = k % _NSLOTS
            return pltpu.make_async_copy(
                buf.at[slot], o1_hbm.at[n, p0 + k], sem_out.at[slot])

        for k in range(min(_NSLOTS, _NCHUNKS)):
            for cp in in_copies(k):
                cp.start()

        for k in range(_NCHUNKS):
            for cp in in_copies(k):
                cp.wait()
            out_copy(k).start()
            j = k - _LAG
            if j >= 0 and j + _NSLOTS < _NCHUNKS:
                out_copy(j).wait()
                for cp in in_copies(j + _NSLOTS):
                    cp.start()

        lo = max(0, min(_NCHUNKS - _LAG, _NCHUNKS - _NSLOTS))
        for k in range(lo, _NCHUNKS):
            out_copy(k).wait()

    return sc_out1


def _tc_out2_body(x1_ref, x2_ref, o2_ref):
    o2_ref[:, :, 0] = x1_ref[:, :, 0]
    o2_ref[:, :, 1] = x2_ref[:, :, 0]


def _make_tc_kernel(dtype):
    grid = (_N, _CPAIRS // _TC_BLOCK)
    return pl.pallas_call(
        _tc_out2_body,
        grid=grid,
        in_specs=[
            pl.BlockSpec((1, _TC_BLOCK, 1, _H, _W), lambda n, b: (n, b, 0, 0, 0)),
            pl.BlockSpec((1, _TC_BLOCK, 1, _H, _W), lambda n, b: (n, b, 1, 0, 0)),
        ],
        out_specs=pl.BlockSpec((1, _TC_BLOCK, 2, _H, _W),
                               lambda n, b: (n, b, 0, 0, 0)),
        out_shape=jax.ShapeDtypeStruct((_N, _CPAIRS, 2, _H, _W), dtype),
    )


def kernel(x1, x2):
    N, c, h, w = x1.shape
    a = x1.reshape(N, c // 2, 2, h, w)
    b = x2.reshape(N, c // 2, 2, h, w)
    o1 = _make_sc_kernel(x1.dtype)(a, b)
    o2 = _make_tc_kernel(x1.dtype)(a, b)
    return (o1.reshape(N, c, h, w), o2.reshape(N, c, h, w))
